# baseline (speedup 1.0000x reference)
import math

import jax
import jax.numpy as jnp
from jax import lax
from jax.experimental import pallas as pl
from jax.experimental.pallas import tpu as pltpu
from jax.experimental.pallas import tpu_sc as plsc

VOCAB = 100000
CTX = 2048
DIM = 768
BATCH = 4
SCALE = math.sqrt(DIM)

ROWS = BATCH * CTX
NW = 32
PPW = CTX // NW
PC = 16
NCHUNK = PPW // PC
NBUF = 2
LANES = 16
VPR = DIM // LANES


def _emb_body(x_hbm, tab_hbm, pos_hbm, out_hbm, idx_stage, idx_v, rows_v,
              pos_v, psem0, psem1, isem, gsem0, gsem1, ssem0, ssem1):
    psems = (psem0, psem1)
    gsems = (gsem0, gsem1)
    ssems = (ssem0, ssem1)
    cid = lax.axis_index("c")
    sid = lax.axis_index("s")
    wid = sid * 2 + cid
    pbase = wid * PPW

    icopies = [
        pltpu.async_copy(
            x_hbm.at[pl.ds(b * CTX + pbase, PPW)],
            idx_stage.at[pl.ds(b * PPW, PPW)], isem)
        for b in range(BATCH)
    ]
    for cp in icopies:
        cp.wait()
    for i in range(BATCH * PPW // LANES):
        kk, b0 = i >> 2, i & 3
        idx_v[pl.ds(i * LANES, LANES)] = (
            idx_stage[pl.ds(b0 * PPW + kk * PC, LANES)])

    def issue(k):
        bsel = k % NBUF
        pltpu.async_copy(
            pos_hbm.at[pl.ds(pbase + k * PC, PC)], pos_v.at[bsel],
            psems[bsel])
        return pltpu.async_copy(
            tab_hbm.at[idx_v.at[pl.ds(k * BATCH * PC, BATCH * PC)]],
            rows_v.at[bsel], gsems[bsel])

    inflight = [None] * NCHUNK
    stores = [None] * NCHUNK
    for d in range(NBUF):
        inflight[d] = issue(d)
    for k in range(NCHUNK):
        bsel = k % NBUF
        inflight[k].wait()
        pltpu.make_async_copy(
            pos_hbm.at[pl.ds(pbase, PC)], pos_v.at[bsel],
            psems[bsel]).wait()
        buf = rows_v.at[bsel]
        pbuf = pos_v.at[bsel]

        @plsc.parallel_loop(0, VPR * PC, 1, unroll=4)
        def _vec_body(i):
            j = i // PC
            p = lax.rem(i, PC)
            sl = pl.ds(j * LANES, LANES)
            pv = pbuf[p, sl]
            for b in range(BATCH):
                buf[b * PC + p, sl] = buf[b * PC + p, sl] * SCALE + pv

        stores[k] = [
            pltpu.async_copy(
                buf.at[pl.ds(b * PC, PC)],
                out_hbm.at[pl.ds(b * CTX + pbase + k * PC, PC)], ssems[bsel])
            for b in range(BATCH)
        ]
        if k + NBUF < NCHUNK:
            for s in stores[k]:
                s.wait()
            inflight[k + NBUF] = issue(k + NBUF)
    for k in range(max(NCHUNK - NBUF, 0), NCHUNK):
        for s in stores[k]:
            s.wait()


def kernel(x, token_emb, pos_enc):
    x_flat = x.reshape(ROWS).astype(jnp.int32)
    pos2d = pos_enc.reshape(CTX, DIM)

    mesh = plsc.VectorSubcoreMesh(core_axis_name="c", subcore_axis_name="s")
    out = pl.kernel(
        _emb_body,
        mesh=mesh,
        out_type=jax.ShapeDtypeStruct((ROWS, DIM), jnp.float32),
        scratch_types=[
            pltpu.VMEM((BATCH * PPW,), jnp.int32),
            pltpu.VMEM((NCHUNK * BATCH * PC,), jnp.int32),
            pltpu.VMEM((NBUF, BATCH * PC, DIM), jnp.float32),
            pltpu.VMEM((NBUF, PC, DIM), jnp.float32),
            pltpu.SemaphoreType.DMA,
            pltpu.SemaphoreType.DMA,
            pltpu.SemaphoreType.DMA,
            pltpu.SemaphoreType.DMA,
            pltpu.SemaphoreType.DMA,
            pltpu.SemaphoreType.DMA,
            pltpu.SemaphoreType.DMA,
        ],
    )(x_flat, token_emb, pos2d)
    return out.reshape(BATCH, CTX, DIM)


# speedup vs baseline: 1.0482x; 1.0482x over previous
import math

import jax
import jax.numpy as jnp
from jax import lax
from jax.experimental import pallas as pl
from jax.experimental.pallas import tpu as pltpu
from jax.experimental.pallas import tpu_sc as plsc

VOCAB = 100000
CTX = 2048
DIM = 768
BATCH = 4
SCALE = math.sqrt(DIM)

ROWS = BATCH * CTX
NW = 32
PPW = CTX // NW
PC = 8
NCHUNK = PPW // PC
NBUF = 3
LANES = 16
VPR = DIM // LANES


def _emb_body(x_hbm, tab_hbm, pos_hbm, out_hbm, idx_stage, idx_v, rows_v,
              pos_v, psem, isem, gsem0, gsem1, gsem2, ssem0, ssem1, ssem2):
    gsems = (gsem0, gsem1, gsem2)
    ssems = (ssem0, ssem1, ssem2)
    cid = lax.axis_index("c")
    sid = lax.axis_index("s")
    wid = sid * 2 + cid
    pbase = wid * PPW

    pos_cp = pltpu.async_copy(pos_hbm.at[pl.ds(pbase, PPW)], pos_v, psem)
    icopies = [
        pltpu.async_copy(
            x_hbm.at[pl.ds(b * CTX + pbase, PPW)],
            idx_stage.at[pl.ds(b * PPW, PPW)], isem)
        for b in range(BATCH)
    ]
    for cp in icopies:
        cp.wait()
    low8 = lax.iota(jnp.int32, LANES) < PC
    for i in range(BATCH * PPW // LANES):
        kk, b0, b1 = i >> 1, (2 * i) & 3, (2 * i + 1) & 3
        va = idx_stage[pl.ds(b0 * PPW + kk * PC, LANES)]
        vb = idx_stage[pl.ds(b1 * PPW + kk * PC - PC, LANES)]
        idx_v[pl.ds(i * LANES, LANES)] = jnp.where(low8, va, vb)

    def issue(k):
        bsel = k % NBUF
        return pltpu.async_copy(
            tab_hbm.at[idx_v.at[pl.ds(k * BATCH * PC, BATCH * PC)]], rows_v.at[bsel], gsems[bsel])

    inflight = [None] * NCHUNK
    stores = [None] * NCHUNK
    for d in range(NBUF):
        inflight[d] = issue(d)
    pos_cp.wait()
    for k in range(NCHUNK):
        bsel = k % NBUF
        inflight[k].wait()
        buf = rows_v.at[bsel]

        @plsc.parallel_loop(0, VPR * PC, 1, unroll=4)
        def _vec_body(i):
            j = i // PC
            p = lax.rem(i, PC)
            sl = pl.ds(j * LANES, LANES)
            pv = pos_v[k * PC + p, sl]
            for b in range(BATCH):
                buf[b * PC + p, sl] = buf[b * PC + p, sl] * SCALE + pv

        stores[k] = [
            pltpu.async_copy(
                buf.at[pl.ds(b * PC, PC)],
                out_hbm.at[pl.ds(b * CTX + pbase + k * PC, PC)], ssems[bsel])
            for b in range(BATCH)
        ]
        if k + NBUF < NCHUNK:
            for s in stores[k]:
                s.wait()
            inflight[k + NBUF] = issue(k + NBUF)
    for k in range(max(NCHUNK - NBUF, 0), NCHUNK):
        for s in stores[k]:
            s.wait()


def kernel(x, token_emb, pos_enc):
    x_flat = x.reshape(ROWS).astype(jnp.int32)
    pos2d = pos_enc.reshape(CTX, DIM)

    mesh = plsc.VectorSubcoreMesh(core_axis_name="c", subcore_axis_name="s")
    out = pl.kernel(
        _emb_body,
        mesh=mesh,
        out_type=jax.ShapeDtypeStruct((ROWS, DIM), jnp.float32),
        scratch_types=[
            pltpu.VMEM((BATCH * PPW,), jnp.int32),
            pltpu.VMEM((NCHUNK * BATCH * PC,), jnp.int32),
            pltpu.VMEM((NBUF, BATCH * PC, DIM), jnp.float32),
            pltpu.VMEM((PPW, DIM), jnp.float32),
            pltpu.SemaphoreType.DMA,
            pltpu.SemaphoreType.DMA,
            pltpu.SemaphoreType.DMA,
            pltpu.SemaphoreType.DMA,
            pltpu.SemaphoreType.DMA,
            pltpu.SemaphoreType.DMA,
            pltpu.SemaphoreType.DMA,
            pltpu.SemaphoreType.DMA,
        ],
    )(x_flat, token_emb, pos2d)
    return out.reshape(BATCH, CTX, DIM)
